# Initial kernel scaffold; baseline (speedup 1.0000x reference)
import jax
import jax.numpy as jnp
from jax.experimental import pallas as pl


def kernel(features, edge_index, Ws0, Wn0, b0, Ws1, Wn1, b1, Ws2, Wn2, b2, g0, be0, g1, be1):
    raise NotImplementedError("write your pallas kernel here")


# speedup vs baseline: 6.8371x; 6.8371x over previous
import functools

import jax
import jax.numpy as jnp
from jax import lax
from jax.experimental import pallas as pl
from jax.experimental.pallas import tpu as pltpu
from jax.experimental.pallas import tpu_sc as plsc

N = 10000
E = 320000
D_IN = 128
D_H = 128
D_OUT = 64
DC = 64

NC = 2
NS = 16
EPW = E // NS
C = 50
NCHUNK = EPW // C
RPS = 640
RPS_LAST = N - (NS - 1) * RPS
ZR = 80
DW = 16

_mesh = plsc.VectorSubcoreMesh(core_axis_name="c", subcore_axis_name="s")


def _agg_body(with_deg, h_hbm, src_hbm, dst_hbm, *rest):
    if with_deg:
        (acc_out, deg_out, srcv, dstv, rows, zbuf, zbuf_d, onesv, sem,
         acc_sh, deg_sh) = rest
    else:
        acc_out, srcv, dstv, rows, zbuf, sem, acc_sh = rest
    cid = lax.axis_index("c")
    sid = lax.axis_index("s")

    nz = jnp.where(sid < NS - 1, RPS // ZR, RPS_LAST // ZR)
    roff = pl.multiple_of(sid * RPS, 8)

    @pl.loop(0, ZR)
    def _(i):
        @pl.loop(0, DC, step=16)
        def _(j):
            zbuf[i, pl.ds(j, 16)] = jnp.zeros((16,), jnp.float32)

    @pl.loop(0, nz)
    def _(k):
        pltpu.sync_copy(zbuf, acc_sh.at[pl.ds(roff + k * ZR, ZR)])

    if with_deg:
        @pl.when(cid == 0)
        def _():
            @pl.loop(0, ZR)
            def _(i):
                zbuf_d[i, pl.ds(0, DW)] = jnp.zeros((DW,), jnp.float32)

            @pl.loop(0, C)
            def _(i):
                onesv[i, pl.ds(0, DW)] = jnp.ones((DW,), jnp.float32)

            @pl.loop(0, nz)
            def _(k):
                pltpu.sync_copy(zbuf_d, deg_sh.at[pl.ds(roff + k * ZR, ZR)])

    plsc.subcore_barrier()

    coff = pl.multiple_of(sid * NCHUNK, 8)
    pltpu.sync_copy(src_hbm.at[pl.ds(coff, NCHUNK)], srcv)
    pltpu.sync_copy(dst_hbm.at[pl.ds(coff, NCHUNK)], dstv)

    h_half = h_hbm.at[cid]

    def gcopy(j, b):
        return pltpu.make_async_copy(h_half.at[srcv.at[j]], rows.at[b], sem)

    def scat(j, b):
        pltpu.sync_copy(rows.at[b], acc_sh.at[dstv.at[j]], add=True)
        if with_deg:
            @pl.when(cid == 0)
            def _():
                pltpu.sync_copy(onesv, deg_sh.at[dstv.at[j]], add=True)

    gcopy(0, 0).start()

    @pl.loop(0, NCHUNK, step=2)
    def _(j):
        gcopy(j + 1, 1).start()
        gcopy(j, 0).wait()
        scat(j, 0)

        @pl.when(j + 2 < NCHUNK)
        def _():
            gcopy(j + 2, 0).start()

        gcopy(j + 1, 1).wait()
        scat(j + 1, 1)

    plsc.subcore_barrier()

    @pl.when(sid < NS - 1)
    def _():
        pltpu.sync_copy(acc_sh.at[pl.ds(roff, RPS)],
                        acc_out.at[cid].at[pl.ds(roff, RPS)])
        if with_deg:
            @pl.when(cid == 0)
            def _():
                pltpu.sync_copy(deg_sh.at[pl.ds(roff, RPS)],
                                deg_out.at[pl.ds(roff, RPS)])

    @pl.when(sid == NS - 1)
    def _():
        pltpu.sync_copy(acc_sh.at[pl.ds(roff, RPS_LAST)],
                        acc_out.at[cid].at[pl.ds(roff, RPS_LAST)])
        if with_deg:
            @pl.when(cid == 0)
            def _():
                pltpu.sync_copy(deg_sh.at[pl.ds(roff, RPS_LAST)],
                                deg_out.at[pl.ds(roff, RPS_LAST)])


def _make_agg(with_deg):
    out_type = [jax.ShapeDtypeStruct((NC, N, DC), jnp.float32)]
    scratch = [
        pltpu.VMEM((NCHUNK, C), jnp.int32),
        pltpu.VMEM((NCHUNK, C), jnp.int32),
        pltpu.VMEM((2, C, DC), jnp.float32),
        pltpu.VMEM((ZR, DC), jnp.float32),
    ]
    if with_deg:
        out_type.append(jax.ShapeDtypeStruct((N, DW), jnp.float32))
        scratch += [
            pltpu.VMEM((ZR, DW), jnp.float32),
            pltpu.VMEM((C, DW), jnp.float32),
        ]
    scratch.append(pltpu.SemaphoreType.DMA)
    scratch.append(pltpu.VMEM_SHARED((N, DC), jnp.float32))
    if with_deg:
        scratch.append(pltpu.VMEM_SHARED((N, DW), jnp.float32))
    return pl.kernel(
        functools.partial(_agg_body, with_deg),
        out_type=tuple(out_type) if len(out_type) > 1 else out_type[0],
        mesh=_mesh,
        scratch_types=scratch,
        compiler_params=pltpu.CompilerParams(use_tc_tiling_on_sc=False),
    )


def _tc0_body(h, a, dg, ws, wn, b, g, be, out, recip_out):
    deg = dg[:, 0:1]
    recip = jnp.where(deg > 0, 1.0 / jnp.maximum(deg, 1.0), 0.0)
    hn = jnp.concatenate([a[0], a[1]], axis=1) * recip
    z = jnp.dot(h[...], ws[...], preferred_element_type=jnp.float32)
    z = z + jnp.dot(hn, wn[...], preferred_element_type=jnp.float32) + b[...]
    z = jnp.maximum(z, 0.0)
    mu = jnp.mean(z, axis=0, keepdims=True)
    var = jnp.mean((z - mu) ** 2, axis=0, keepdims=True)
    z = (z - mu) * lax.rsqrt(var + 1e-5) * g[...] + be[...]
    out[...] = jnp.maximum(z, 0.0)
    recip_out[...] = recip


def _tc1_body(h, a, recip, ws, wn, b, g, be, out):
    hn = jnp.concatenate([a[0], a[1]], axis=1) * recip[...]
    z = jnp.dot(h[...], ws[...], preferred_element_type=jnp.float32)
    z = z + jnp.dot(hn, wn[...], preferred_element_type=jnp.float32) + b[...]
    z = jnp.maximum(z, 0.0)
    mu = jnp.mean(z, axis=0, keepdims=True)
    var = jnp.mean((z - mu) ** 2, axis=0, keepdims=True)
    z = (z - mu) * lax.rsqrt(var + 1e-5) * g[...] + be[...]
    out[...] = jnp.maximum(z, 0.0)


def _tc2_body(h, a, recip, ws, wn, b, out):
    hn = jnp.concatenate([a[0], a[1]], axis=1) * recip[...]
    z = jnp.dot(h[...], ws[...], preferred_element_type=jnp.float32)
    z = z + jnp.dot(hn, wn[...], preferred_element_type=jnp.float32) + b[...]
    hg = jnp.mean(z, axis=0, keepdims=True)
    m = jnp.max(hg, axis=1, keepdims=True)
    e = jnp.exp(hg - m)
    out[...] = e / jnp.sum(e, axis=1, keepdims=True)


_agg_with_deg = _make_agg(True)
_agg_plain = _make_agg(False)

_tc0 = pl.pallas_call(
    _tc0_body,
    out_shape=(jax.ShapeDtypeStruct((N, D_H), jnp.float32),
               jax.ShapeDtypeStruct((N, 1), jnp.float32)),
)

_tc1 = pl.pallas_call(
    _tc1_body,
    out_shape=jax.ShapeDtypeStruct((N, D_H), jnp.float32),
)

_tc2 = pl.pallas_call(
    _tc2_body,
    out_shape=jax.ShapeDtypeStruct((1, D_OUT), jnp.float32),
)


def _halves(h):
    return h.reshape(N, NC, DC).transpose(1, 0, 2)


def kernel(features, edge_index, Ws0, Wn0, b0, Ws1, Wn1, b1, Ws2, Wn2, b2,
           g0, be0, g1, be1):
    src2d = edge_index[0].reshape(E // C, C)
    dst2d = edge_index[1].reshape(E // C, C)

    acc0, deg = _agg_with_deg(_halves(features), src2d, dst2d)
    h1, recip = _tc0(features, acc0, deg, Ws0, Wn0, b0.reshape(1, D_H),
                     g0.reshape(1, D_H), be0.reshape(1, D_H))

    acc1 = _agg_plain(_halves(h1), src2d, dst2d)
    h2 = _tc1(h1, acc1, recip, Ws1, Wn1, b1.reshape(1, D_H),
              g1.reshape(1, D_H), be1.reshape(1, D_H))

    acc2 = _agg_plain(_halves(h2), src2d, dst2d)
    return _tc2(h2, acc2, recip, Ws2, Wn2, b2.reshape(1, D_OUT))


# speedup vs baseline: 10.4980x; 1.5354x over previous
import functools

import jax
import jax.numpy as jnp
from jax import lax
from jax.experimental import pallas as pl
from jax.experimental.pallas import tpu as pltpu
from jax.experimental.pallas import tpu_sc as plsc

N = 10000
E = 320000
D_IN = 128
D_H = 128
D_OUT = 64

NC = 2
NS = 16
EPW = E // NS
C = 125
NCHUNK = EPW // C
RPS = 640
RPS_LAST = N - (NS - 1) * RPS
ZR = 80
DW = 16

_mesh = plsc.VectorSubcoreMesh(core_axis_name="c", subcore_axis_name="s")


def _agg_body(with_deg, dc, h_hbm, src_hbm, dst_hbm, *rest):
    if with_deg:
        (acc_out, deg_out, srcv, dstv, rows, zbuf, zbuf_d, onesv, sem,
         acc_sh, deg_sh) = rest
    else:
        acc_out, srcv, dstv, rows, zbuf, sem, acc_sh = rest
    cid = lax.axis_index("c")
    sid = lax.axis_index("s")

    nz = jnp.where(sid < NS - 1, RPS // ZR, RPS_LAST // ZR)
    roff = pl.multiple_of(sid * RPS, 8)

    @pl.loop(0, ZR)
    def _(i):
        @pl.loop(0, dc, step=16)
        def _(j):
            zbuf[i, pl.ds(j, 16)] = jnp.zeros((16,), jnp.float32)

    @pl.loop(0, nz)
    def _(k):
        pltpu.sync_copy(zbuf, acc_sh.at[pl.ds(roff + k * ZR, ZR)])

    if with_deg:
        @pl.when(cid == 0)
        def _():
            @pl.loop(0, ZR)
            def _(i):
                zbuf_d[i, pl.ds(0, DW)] = jnp.zeros((DW,), jnp.float32)

            @pl.loop(0, C)
            def _(i):
                onesv[i, pl.ds(0, DW)] = jnp.ones((DW,), jnp.float32)

            @pl.loop(0, nz)
            def _(k):
                pltpu.sync_copy(zbuf_d, deg_sh.at[pl.ds(roff + k * ZR, ZR)])

    plsc.subcore_barrier()

    coff = pl.multiple_of(sid * NCHUNK, 8)
    pltpu.sync_copy(src_hbm.at[pl.ds(coff, NCHUNK)], srcv)
    pltpu.sync_copy(dst_hbm.at[pl.ds(coff, NCHUNK)], dstv)

    h_half = h_hbm.at[cid]

    def gcopy(j, b):
        return pltpu.make_async_copy(h_half.at[srcv.at[j]], rows.at[b], sem)

    def scat(j, b):
        pltpu.sync_copy(rows.at[b], acc_sh.at[dstv.at[j]], add=True)
        if with_deg:
            @pl.when(cid == 0)
            def _():
                pltpu.sync_copy(onesv, deg_sh.at[dstv.at[j]], add=True)

    gcopy(0, 0).start()

    @pl.loop(0, NCHUNK, step=2)
    def _(j):
        gcopy(j + 1, 1).start()
        gcopy(j, 0).wait()
        scat(j, 0)

        @pl.when(j + 2 < NCHUNK)
        def _():
            gcopy(j + 2, 0).start()

        gcopy(j + 1, 1).wait()
        scat(j + 1, 1)

    plsc.subcore_barrier()

    @pl.when(sid < NS - 1)
    def _():
        pltpu.sync_copy(acc_sh.at[pl.ds(roff, RPS)],
                        acc_out.at[cid].at[pl.ds(roff, RPS)])
        if with_deg:
            @pl.when(cid == 0)
            def _():
                pltpu.sync_copy(deg_sh.at[pl.ds(roff, RPS)],
                                deg_out.at[pl.ds(roff, RPS)])

    @pl.when(sid == NS - 1)
    def _():
        pltpu.sync_copy(acc_sh.at[pl.ds(roff, RPS_LAST)],
                        acc_out.at[cid].at[pl.ds(roff, RPS_LAST)])
        if with_deg:
            @pl.when(cid == 0)
            def _():
                pltpu.sync_copy(deg_sh.at[pl.ds(roff, RPS_LAST)],
                                deg_out.at[pl.ds(roff, RPS_LAST)])


def _make_agg(with_deg, dc):
    out_type = [jax.ShapeDtypeStruct((NC, N, dc), jnp.float32)]
    scratch = [
        pltpu.VMEM((NCHUNK, C), jnp.int32),
        pltpu.VMEM((NCHUNK, C), jnp.int32),
        pltpu.VMEM((2, C, dc), jnp.float32),
        pltpu.VMEM((ZR, dc), jnp.float32),
    ]
    if with_deg:
        out_type.append(jax.ShapeDtypeStruct((N, DW), jnp.float32))
        scratch += [
            pltpu.VMEM((ZR, DW), jnp.float32),
            pltpu.VMEM((C, DW), jnp.float32),
        ]
    scratch.append(pltpu.SemaphoreType.DMA)
    scratch.append(pltpu.VMEM_SHARED((N, dc), jnp.float32))
    if with_deg:
        scratch.append(pltpu.VMEM_SHARED((N, DW), jnp.float32))
    return pl.kernel(
        functools.partial(_agg_body, with_deg, dc),
        out_type=tuple(out_type) if len(out_type) > 1 else out_type[0],
        mesh=_mesh,
        scratch_types=scratch,
        compiler_params=pltpu.CompilerParams(use_tc_tiling_on_sc=False),
    )


def _self_body(h, ws, b, out):
    out[...] = jnp.dot(h[...], ws[...],
                       preferred_element_type=jnp.float32) + b[...]


def _matmul_body(h, w, out):
    out[...] = jnp.dot(h[...], w[...], preferred_element_type=jnp.float32)


def _combine0_body(zs, a, dg, wn, g, be, out, recip_out):
    deg = dg[:, 0:1]
    recip = jnp.where(deg > 0, 1.0 / jnp.maximum(deg, 1.0), 0.0)
    hn = jnp.concatenate([a[0], a[1]], axis=1) * recip
    z = zs[...] + jnp.dot(hn, wn[...], preferred_element_type=jnp.float32)
    z = jnp.maximum(z, 0.0)
    mu = jnp.mean(z, axis=0, keepdims=True)
    var = jnp.mean((z - mu) ** 2, axis=0, keepdims=True)
    z = (z - mu) * lax.rsqrt(var + 1e-5) * g[...] + be[...]
    out[...] = jnp.maximum(z, 0.0)
    recip_out[...] = recip


def _combine1_body(zs, a, recip, wn, g, be, out):
    hn = jnp.concatenate([a[0], a[1]], axis=1) * recip[...]
    z = zs[...] + jnp.dot(hn, wn[...], preferred_element_type=jnp.float32)
    z = jnp.maximum(z, 0.0)
    mu = jnp.mean(z, axis=0, keepdims=True)
    var = jnp.mean((z - mu) ** 2, axis=0, keepdims=True)
    z = (z - mu) * lax.rsqrt(var + 1e-5) * g[...] + be[...]
    out[...] = jnp.maximum(z, 0.0)


def _final_body(zs, a, recip, out):
    z = zs[...] + jnp.concatenate([a[0], a[1]], axis=1) * recip[...]
    hg = jnp.mean(z, axis=0, keepdims=True)
    m = jnp.max(hg, axis=1, keepdims=True)
    e = jnp.exp(hg - m)
    out[...] = e / jnp.sum(e, axis=1, keepdims=True)


_agg_with_deg = _make_agg(True, 64)
_agg_plain = _make_agg(False, 64)
_agg_out = _make_agg(False, 32)

_f32 = jnp.float32
_self128 = pl.pallas_call(
    _self_body, out_shape=jax.ShapeDtypeStruct((N, D_H), _f32))
_self64 = pl.pallas_call(
    _self_body, out_shape=jax.ShapeDtypeStruct((N, D_OUT), _f32))
_mm64 = pl.pallas_call(
    _matmul_body, out_shape=jax.ShapeDtypeStruct((N, D_OUT), _f32))
_combine0 = pl.pallas_call(
    _combine0_body,
    out_shape=(jax.ShapeDtypeStruct((N, D_H), _f32),
               jax.ShapeDtypeStruct((N, 1), _f32)))
_combine1 = pl.pallas_call(
    _combine1_body, out_shape=jax.ShapeDtypeStruct((N, D_H), _f32))
_final = pl.pallas_call(
    _final_body, out_shape=jax.ShapeDtypeStruct((1, D_OUT), _f32))


def _halves(h, dc):
    return h.reshape(N, NC, dc).transpose(1, 0, 2)


def kernel(features, edge_index, Ws0, Wn0, b0, Ws1, Wn1, b1, Ws2, Wn2, b2,
           g0, be0, g1, be1):
    src2d = edge_index[0].reshape(E // C, C)
    dst2d = edge_index[1].reshape(E // C, C)

    acc0, deg = _agg_with_deg(_halves(features, 64), src2d, dst2d)
    zs0 = _self128(features, Ws0, b0.reshape(1, D_H))
    h1, recip = _combine0(zs0, acc0, deg, Wn0,
                          g0.reshape(1, D_H), be0.reshape(1, D_H))

    acc1 = _agg_plain(_halves(h1, 64), src2d, dst2d)
    zs1 = _self128(h1, Ws1, b1.reshape(1, D_H))
    h2 = _combine1(zs1, acc1, recip, Wn1,
                   g1.reshape(1, D_H), be1.reshape(1, D_H))

    y2 = _mm64(h2, Wn2)
    acc2 = _agg_out(_halves(y2, 32), src2d, dst2d)
    zs2 = _self64(h2, Ws2, b2.reshape(1, D_OUT))
    return _final(zs2, acc2, recip)


# speedup vs baseline: 11.0829x; 1.0557x over previous
import functools

import jax
import jax.numpy as jnp
from jax import lax
from jax.experimental import pallas as pl
from jax.experimental.pallas import tpu as pltpu
from jax.experimental.pallas import tpu_sc as plsc

N = 10000
E = 320000
D_IN = 128
D_H = 128
D_OUT = 64

NC = 2
NS = 16
EPW = E // NS
C = 125
NCHUNK = EPW // C
RPS = 640
RPS_LAST = N - (NS - 1) * RPS
ZR = 80
DW = 16

_mesh = plsc.VectorSubcoreMesh(core_axis_name="c", subcore_axis_name="s")


def _agg_body(with_deg, dc, h_hbm, src_hbm, dst_hbm, *rest):
    if with_deg:
        (acc_out, deg_out, srcv, dstv, rows, zbuf, zbuf_d, onesv, sem,
         acc_sh, deg_sh) = rest
    else:
        acc_out, srcv, dstv, rows, zbuf, sem, acc_sh = rest
    cid = lax.axis_index("c")
    sid = lax.axis_index("s")

    nz = jnp.where(sid < NS - 1, RPS // ZR, RPS_LAST // ZR)
    roff = pl.multiple_of(sid * RPS, 8)

    @pl.loop(0, ZR)
    def _(i):
        @pl.loop(0, dc, step=16)
        def _(j):
            zbuf[i, pl.ds(j, 16)] = jnp.zeros((16,), jnp.float32)

    @pl.loop(0, nz)
    def _(k):
        pltpu.sync_copy(zbuf, acc_sh.at[pl.ds(roff + k * ZR, ZR)])

    if with_deg:
        @pl.when(cid == 0)
        def _():
            @pl.loop(0, ZR)
            def _(i):
                zbuf_d[i, pl.ds(0, DW)] = jnp.zeros((DW,), jnp.float32)

            @pl.loop(0, C)
            def _(i):
                onesv[i, pl.ds(0, DW)] = jnp.ones((DW,), jnp.float32)

            @pl.loop(0, nz)
            def _(k):
                pltpu.sync_copy(zbuf_d, deg_sh.at[pl.ds(roff + k * ZR, ZR)])

    plsc.subcore_barrier()

    coff = pl.multiple_of(sid * NCHUNK, 8)
    pltpu.sync_copy(src_hbm.at[pl.ds(coff, NCHUNK)], srcv)
    pltpu.sync_copy(dst_hbm.at[pl.ds(coff, NCHUNK)], dstv)

    h_half = h_hbm.at[cid]

    def gcopy(j, b):
        return pltpu.make_async_copy(h_half.at[srcv.at[j]], rows.at[b], sem)

    def scat(j, b):
        pltpu.sync_copy(rows.at[b], acc_sh.at[dstv.at[j]], add=True)
        if with_deg:
            @pl.when(cid == 0)
            def _():
                pltpu.sync_copy(onesv, deg_sh.at[dstv.at[j]], add=True)

    gcopy(0, 0).start()

    @pl.loop(0, NCHUNK, step=2)
    def _(j):
        gcopy(j + 1, 1).start()
        gcopy(j, 0).wait()
        scat(j, 0)

        @pl.when(j + 2 < NCHUNK)
        def _():
            gcopy(j + 2, 0).start()

        gcopy(j + 1, 1).wait()
        scat(j + 1, 1)

    plsc.subcore_barrier()

    @pl.when(sid < NS - 1)
    def _():
        pltpu.sync_copy(acc_sh.at[pl.ds(roff, RPS)],
                        acc_out.at[cid].at[pl.ds(roff, RPS)])
        if with_deg:
            @pl.when(cid == 0)
            def _():
                pltpu.sync_copy(deg_sh.at[pl.ds(roff, RPS)],
                                deg_out.at[pl.ds(roff, RPS)])

    @pl.when(sid == NS - 1)
    def _():
        pltpu.sync_copy(acc_sh.at[pl.ds(roff, RPS_LAST)],
                        acc_out.at[cid].at[pl.ds(roff, RPS_LAST)])
        if with_deg:
            @pl.when(cid == 0)
            def _():
                pltpu.sync_copy(deg_sh.at[pl.ds(roff, RPS_LAST)],
                                deg_out.at[pl.ds(roff, RPS_LAST)])


def _make_agg(with_deg, dc):
    out_type = [jax.ShapeDtypeStruct((NC, N, dc), jnp.float32)]
    scratch = [
        pltpu.VMEM((NCHUNK, C), jnp.int32),
        pltpu.VMEM((NCHUNK, C), jnp.int32),
        pltpu.VMEM((2, C, dc), jnp.float32),
        pltpu.VMEM((ZR, dc), jnp.float32),
    ]
    if with_deg:
        out_type.append(jax.ShapeDtypeStruct((N, DW), jnp.float32))
        scratch += [
            pltpu.VMEM((ZR, DW), jnp.float32),
            pltpu.VMEM((C, DW), jnp.float32),
        ]
    scratch.append(pltpu.SemaphoreType.DMA)
    scratch.append(pltpu.VMEM_SHARED((N, dc), jnp.float32))
    if with_deg:
        scratch.append(pltpu.VMEM_SHARED((N, DW), jnp.float32))
    return pl.kernel(
        functools.partial(_agg_body, with_deg, dc),
        out_type=tuple(out_type) if len(out_type) > 1 else out_type[0],
        mesh=_mesh,
        scratch_types=scratch,
        compiler_params=pltpu.CompilerParams(use_tc_tiling_on_sc=False),
    )


def _split_body(h, out):
    out[...] = jnp.stack([h[:, :64], h[:, 64:]], axis=0)


def _bn_relu(z, g, be):
    mu = jnp.mean(z, axis=0, keepdims=True)
    var = jnp.mean((z - mu) ** 2, axis=0, keepdims=True)
    z = (z - mu) * lax.rsqrt(var + 1e-5) * g + be
    return jnp.maximum(z, 0.0)


def _combine0_body(h, a, dg, ws0, wn0, b0, g0, be0, ws1, b1,
                   h1s_out, zs1_out, recip_out):
    deg = dg[:, 0:1]
    recip = jnp.where(deg > 0, 1.0 / jnp.maximum(deg, 1.0), 0.0)
    hn = jnp.concatenate([a[0], a[1]], axis=1) * recip
    z = jnp.dot(h[...], ws0[...], preferred_element_type=jnp.float32)
    z = z + jnp.dot(hn, wn0[...], preferred_element_type=jnp.float32) + b0[...]
    h1 = _bn_relu(jnp.maximum(z, 0.0), g0[...], be0[...])
    h1s_out[...] = jnp.stack([h1[:, :64], h1[:, 64:]], axis=0)
    zs1_out[...] = jnp.dot(h1, ws1[...],
                           preferred_element_type=jnp.float32) + b1[...]
    recip_out[...] = recip


def _combine1_body(zs1, a, recip, wn1, g1, be1, wn2, ws2, b2,
                   y2s_out, zs2_out):
    hn = jnp.concatenate([a[0], a[1]], axis=1) * recip[...]
    z = zs1[...] + jnp.dot(hn, wn1[...], preferred_element_type=jnp.float32)
    h2 = _bn_relu(jnp.maximum(z, 0.0), g1[...], be1[...])
    y2 = jnp.dot(h2, wn2[...], preferred_element_type=jnp.float32)
    y2s_out[...] = jnp.stack([y2[:, :32], y2[:, 32:]], axis=0)
    zs2_out[...] = jnp.dot(h2, ws2[...],
                           preferred_element_type=jnp.float32) + b2[...]


def _final_body(zs, a, recip, out):
    z = zs[...] + jnp.concatenate([a[0], a[1]], axis=1) * recip[...]
    hg = jnp.mean(z, axis=0, keepdims=True)
    m = jnp.max(hg, axis=1, keepdims=True)
    e = jnp.exp(hg - m)
    out[...] = e / jnp.sum(e, axis=1, keepdims=True)


_agg_with_deg = _make_agg(True, 64)
_agg_plain = _make_agg(False, 64)
_agg_out = _make_agg(False, 32)

_f32 = jnp.float32
_split = pl.pallas_call(
    _split_body, out_shape=jax.ShapeDtypeStruct((NC, N, 64), _f32))
_combine0 = pl.pallas_call(
    _combine0_body,
    out_shape=(jax.ShapeDtypeStruct((NC, N, 64), _f32),
               jax.ShapeDtypeStruct((N, D_H), _f32),
               jax.ShapeDtypeStruct((N, 1), _f32)))
_combine1 = pl.pallas_call(
    _combine1_body,
    out_shape=(jax.ShapeDtypeStruct((NC, N, 32), _f32),
               jax.ShapeDtypeStruct((N, D_OUT), _f32)))
_final = pl.pallas_call(
    _final_body, out_shape=jax.ShapeDtypeStruct((1, D_OUT), _f32))


def kernel(features, edge_index, Ws0, Wn0, b0, Ws1, Wn1, b1, Ws2, Wn2, b2,
           g0, be0, g1, be1):
    src2d = edge_index[0].reshape(E // C, C)
    dst2d = edge_index[1].reshape(E // C, C)

    acc0, deg = _agg_with_deg(_split(features), src2d, dst2d)
    h1s, zs1, recip = _combine0(features, acc0, deg, Ws0, Wn0,
                                b0.reshape(1, D_H), g0.reshape(1, D_H),
                                be0.reshape(1, D_H), Ws1, b1.reshape(1, D_H))

    acc1 = _agg_plain(h1s, src2d, dst2d)
    y2s, zs2 = _combine1(zs1, acc1, recip, Wn1, g1.reshape(1, D_H),
                         be1.reshape(1, D_H), Wn2, Ws2,
                         b2.reshape(1, D_OUT))

    acc2 = _agg_out(y2s, src2d, dst2d)
    return _final(zs2, acc2, recip)


# speedup vs baseline: 11.2099x; 1.0115x over previous
import functools

import jax
import jax.numpy as jnp
from jax import lax
from jax.experimental import pallas as pl
from jax.experimental.pallas import tpu as pltpu
from jax.experimental.pallas import tpu_sc as plsc

N = 10000
E = 320000
D_IN = 128
D_H = 128
D_OUT = 64

NC = 2
NS = 16
EPW = E // NS
C = 125
NCHUNK = EPW // C
RPS = 640
RPS_LAST = N - (NS - 1) * RPS
ZR = 80
DW = 16

_mesh = plsc.VectorSubcoreMesh(core_axis_name="c", subcore_axis_name="s")


def _agg_body(with_deg, dc, nbuf, h_hbm, src_hbm, dst_hbm, *rest):
    if with_deg:
        (acc_out, deg_out, srcv, dstv, rows, zbuf, zbuf_d, onesv,
         gsem, ssem, acc_sh, deg_sh) = rest
    else:
        acc_out, srcv, dstv, rows, zbuf, gsem, ssem, acc_sh = rest
    cid = lax.axis_index("c")
    sid = lax.axis_index("s")

    nz = jnp.where(sid < NS - 1, RPS // ZR, RPS_LAST // ZR)
    roff = pl.multiple_of(sid * RPS, 8)

    @pl.loop(0, ZR)
    def _(i):
        @pl.loop(0, dc, step=16)
        def _(j):
            zbuf[i, pl.ds(j, 16)] = jnp.zeros((16,), jnp.float32)

    @pl.loop(0, nz)
    def _(k):
        pltpu.sync_copy(zbuf, acc_sh.at[pl.ds(roff + k * ZR, ZR)])

    if with_deg:
        @pl.when(cid == 0)
        def _():
            @pl.loop(0, ZR)
            def _(i):
                zbuf_d[i, pl.ds(0, DW)] = jnp.zeros((DW,), jnp.float32)

            @pl.loop(0, C)
            def _(i):
                onesv[i, pl.ds(0, DW)] = jnp.ones((DW,), jnp.float32)

            @pl.loop(0, nz)
            def _(k):
                pltpu.sync_copy(zbuf_d, deg_sh.at[pl.ds(roff + k * ZR, ZR)])

    plsc.subcore_barrier()

    coff = pl.multiple_of(sid * NCHUNK, 8)
    pltpu.sync_copy(src_hbm.at[pl.ds(coff, NCHUNK)], srcv)
    pltpu.sync_copy(dst_hbm.at[pl.ds(coff, NCHUNK)], dstv)

    h_half = h_hbm.at[cid]

    def gcopy(j, b):
        return pltpu.make_async_copy(h_half.at[srcv.at[j]], rows.at[b],
                                     gsem.at[b])

    def scopy(j, b):
        return pltpu.make_async_copy(rows.at[b], acc_sh.at[dstv.at[j]],
                                     ssem.at[b])

    for b in range(nbuf):
        gcopy(b, b).start()

    @pl.loop(0, NCHUNK, step=nbuf)
    def _(j):
        for b in range(nbuf):
            gcopy(j + b, b).wait()
            pltpu.async_copy(rows.at[b], acc_sh.at[dstv.at[j + b]],
                             ssem.at[b], add=True)
            if with_deg:
                @pl.when(cid == 0)
                def _():
                    pltpu.sync_copy(onesv, deg_sh.at[dstv.at[j + b]],
                                    add=True)
        for b in range(nbuf):
            scopy(j + b, b).wait()

            @pl.when(j + nbuf + b < NCHUNK)
            def _(b=b):
                gcopy(j + nbuf + b, b).start()

    plsc.subcore_barrier()

    @pl.when(sid < NS - 1)
    def _():
        pltpu.sync_copy(acc_sh.at[pl.ds(roff, RPS)],
                        acc_out.at[cid].at[pl.ds(roff, RPS)])
        if with_deg:
            @pl.when(cid == 0)
            def _():
                pltpu.sync_copy(deg_sh.at[pl.ds(roff, RPS)],
                                deg_out.at[pl.ds(roff, RPS)])

    @pl.when(sid == NS - 1)
    def _():
        pltpu.sync_copy(acc_sh.at[pl.ds(roff, RPS_LAST)],
                        acc_out.at[cid].at[pl.ds(roff, RPS_LAST)])
        if with_deg:
            @pl.when(cid == 0)
            def _():
                pltpu.sync_copy(deg_sh.at[pl.ds(roff, RPS_LAST)],
                                deg_out.at[pl.ds(roff, RPS_LAST)])


def _make_agg(with_deg, dc, nbuf):
    out_type = [jax.ShapeDtypeStruct((NC, N, dc), jnp.float32)]
    scratch = [
        pltpu.VMEM((NCHUNK, C), jnp.int32),
        pltpu.VMEM((NCHUNK, C), jnp.int32),
        pltpu.VMEM((nbuf, C, dc), jnp.float32),
        pltpu.VMEM((ZR, dc), jnp.float32),
    ]
    if with_deg:
        out_type.append(jax.ShapeDtypeStruct((N, DW), jnp.float32))
        scratch += [
            pltpu.VMEM((ZR, DW), jnp.float32),
            pltpu.VMEM((C, DW), jnp.float32),
        ]
    scratch.append(pltpu.SemaphoreType.DMA((nbuf,)))
    scratch.append(pltpu.SemaphoreType.DMA((nbuf,)))
    scratch.append(pltpu.VMEM_SHARED((N, dc), jnp.float32))
    if with_deg:
        scratch.append(pltpu.VMEM_SHARED((N, DW), jnp.float32))
    return pl.kernel(
        functools.partial(_agg_body, with_deg, dc, nbuf),
        out_type=tuple(out_type) if len(out_type) > 1 else out_type[0],
        mesh=_mesh,
        scratch_types=scratch,
        compiler_params=pltpu.CompilerParams(use_tc_tiling_on_sc=False),
    )


def _split_body(h, out):
    out[...] = jnp.stack([h[:, :64], h[:, 64:]], axis=0)


def _bn_relu(z, g, be):
    mu = jnp.mean(z, axis=0, keepdims=True)
    var = jnp.mean((z - mu) ** 2, axis=0, keepdims=True)
    z = (z - mu) * lax.rsqrt(var + 1e-5) * g + be
    return jnp.maximum(z, 0.0)


def _combine0_body(h, a, dg, ws0, wn0, b0, g0, be0, ws1, b1,
                   h1s_out, zs1_out, recip_out):
    deg = dg[:, 0:1]
    recip = jnp.where(deg > 0, 1.0 / jnp.maximum(deg, 1.0), 0.0)
    hn = jnp.concatenate([a[0], a[1]], axis=1) * recip
    z = jnp.dot(h[...], ws0[...], preferred_element_type=jnp.float32)
    z = z + jnp.dot(hn, wn0[...], preferred_element_type=jnp.float32) + b0[...]
    h1 = _bn_relu(jnp.maximum(z, 0.0), g0[...], be0[...])
    h1s_out[...] = jnp.stack([h1[:, :64], h1[:, 64:]], axis=0)
    zs1_out[...] = jnp.dot(h1, ws1[...],
                           preferred_element_type=jnp.float32) + b1[...]
    recip_out[...] = recip


def _combine1_body(zs1, a, recip, wn1, g1, be1, wn2, ws2, b2,
                   y2s_out, zs2_out):
    hn = jnp.concatenate([a[0], a[1]], axis=1) * recip[...]
    z = zs1[...] + jnp.dot(hn, wn1[...], preferred_element_type=jnp.float32)
    h2 = _bn_relu(jnp.maximum(z, 0.0), g1[...], be1[...])
    y2 = jnp.dot(h2, wn2[...], preferred_element_type=jnp.float32)
    y2s_out[...] = jnp.stack([y2[:, :32], y2[:, 32:]], axis=0)
    zs2_out[...] = jnp.dot(h2, ws2[...],
                           preferred_element_type=jnp.float32) + b2[...]


def _final_body(zs, a, recip, out):
    z = zs[...] + jnp.concatenate([a[0], a[1]], axis=1) * recip[...]
    hg = jnp.mean(z, axis=0, keepdims=True)
    m = jnp.max(hg, axis=1, keepdims=True)
    e = jnp.exp(hg - m)
    out[...] = e / jnp.sum(e, axis=1, keepdims=True)


_agg_with_deg = _make_agg(True, 64, 2)
_agg_plain = _make_agg(False, 64, 4)
_agg_out = _make_agg(False, 32, 4)

_f32 = jnp.float32
_split = pl.pallas_call(
    _split_body, out_shape=jax.ShapeDtypeStruct((NC, N, 64), _f32))
_combine0 = pl.pallas_call(
    _combine0_body,
    out_shape=(jax.ShapeDtypeStruct((NC, N, 64), _f32),
               jax.ShapeDtypeStruct((N, D_H), _f32),
               jax.ShapeDtypeStruct((N, 1), _f32)))
_combine1 = pl.pallas_call(
    _combine1_body,
    out_shape=(jax.ShapeDtypeStruct((NC, N, 32), _f32),
               jax.ShapeDtypeStruct((N, D_OUT), _f32)))
_final = pl.pallas_call(
    _final_body, out_shape=jax.ShapeDtypeStruct((1, D_OUT), _f32))


def kernel(features, edge_index, Ws0, Wn0, b0, Ws1, Wn1, b1, Ws2, Wn2, b2,
           g0, be0, g1, be1):
    src2d = edge_index[0].reshape(E // C, C)
    dst2d = edge_index[1].reshape(E // C, C)

    acc0, deg = _agg_with_deg(_split(features), src2d, dst2d)
    h1s, zs1, recip = _combine0(features, acc0, deg, Ws0, Wn0,
                                b0.reshape(1, D_H), g0.reshape(1, D_H),
                                be0.reshape(1, D_H), Ws1, b1.reshape(1, D_H))

    acc1 = _agg_plain(h1s, src2d, dst2d)
    y2s, zs2 = _combine1(zs1, acc1, recip, Wn1, g1.reshape(1, D_H),
                         be1.reshape(1, D_H), Wn2, Ws2,
                         b2.reshape(1, D_OUT))

    acc2 = _agg_out(y2s, src2d, dst2d)
    return _final(zs2, acc2, recip)


# speedup vs baseline: 12.2933x; 1.0967x over previous
import functools

import jax
import jax.numpy as jnp
from jax import lax
from jax.experimental import pallas as pl
from jax.experimental.pallas import tpu as pltpu
from jax.experimental.pallas import tpu_sc as plsc

N = 10000
E = 320000
D_IN = 128
D_H = 128
D_OUT = 64

NC = 2
NS = 16
EPW = E // NS
C = 125
NCHUNK = EPW // C
RPS = 640
RPS_LAST = N - (NS - 1) * RPS
ZR = 80
DW = 16

_mesh = plsc.VectorSubcoreMesh(core_axis_name="c", subcore_axis_name="s")


def _agg_body(with_deg, dc, nbuf, h_hbm, src_hbm, dst_hbm, *rest):
    if with_deg:
        (acc_out, deg_out, srcv, dstv, rows, zbuf, zbuf_d, onesv,
         gsem, ssem, dsem, acc_sh, deg_sh) = rest
    else:
        acc_out, srcv, dstv, rows, zbuf, gsem, ssem, acc_sh = rest
    cid = lax.axis_index("c")
    sid = lax.axis_index("s")

    nz = jnp.where(sid < NS - 1, RPS // ZR, RPS_LAST // ZR)
    roff = pl.multiple_of(sid * RPS, 8)

    @pl.loop(0, ZR)
    def _(i):
        @pl.loop(0, dc, step=16)
        def _(j):
            zbuf[i, pl.ds(j, 16)] = jnp.zeros((16,), jnp.float32)

    @pl.loop(0, nz)
    def _(k):
        pltpu.sync_copy(zbuf, acc_sh.at[pl.ds(roff + k * ZR, ZR)])

    if with_deg:
        @pl.when(cid == 0)
        def _():
            @pl.loop(0, ZR)
            def _(i):
                zbuf_d[i, pl.ds(0, DW)] = jnp.zeros((DW,), jnp.float32)

            @pl.loop(0, C)
            def _(i):
                onesv[i, pl.ds(0, DW)] = jnp.ones((DW,), jnp.float32)

            @pl.loop(0, nz)
            def _(k):
                pltpu.sync_copy(zbuf_d, deg_sh.at[pl.ds(roff + k * ZR, ZR)])

    plsc.subcore_barrier()

    coff = pl.multiple_of(sid * NCHUNK, 8)
    pltpu.sync_copy(src_hbm.at[cid].at[pl.ds(coff, NCHUNK)], srcv)
    pltpu.sync_copy(dst_hbm.at[pl.ds(coff, NCHUNK)], dstv)

    def gcopy(j, b):
        return pltpu.make_async_copy(h_hbm.at[srcv.at[j]], rows.at[b],
                                     gsem.at[b])

    def scopy(j, b):
        return pltpu.make_async_copy(rows.at[b], acc_sh.at[dstv.at[j]],
                                     ssem.at[b])

    for b in range(nbuf):
        gcopy(b, b).start()

    @pl.loop(0, NCHUNK, step=nbuf)
    def _(j):
        for b in range(nbuf):
            gcopy(j + b, b).wait()
            pltpu.async_copy(rows.at[b], acc_sh.at[dstv.at[j + b]],
                             ssem.at[b], add=True)
            if with_deg:
                @pl.when(cid == 0)
                def _():
                    pltpu.async_copy(onesv, deg_sh.at[dstv.at[j + b]],
                                     dsem, add=True)
        for b in range(nbuf):
            scopy(j + b, b).wait()

            @pl.when(j + nbuf + b < NCHUNK)
            def _(b=b):
                gcopy(j + nbuf + b, b).start()

    if with_deg:
        @pl.when(cid == 0)
        def _():
            @pl.loop(0, NCHUNK)
            def _(j):
                pltpu.make_async_copy(onesv, deg_sh.at[dstv.at[j]],
                                      dsem).wait()

    plsc.subcore_barrier()

    @pl.when(sid < NS - 1)
    def _():
        pltpu.sync_copy(acc_sh.at[pl.ds(roff, RPS)],
                        acc_out.at[cid].at[pl.ds(roff, RPS)])
        if with_deg:
            @pl.when(cid == 0)
            def _():
                pltpu.sync_copy(deg_sh.at[pl.ds(roff, RPS)],
                                deg_out.at[pl.ds(roff, RPS)])

    @pl.when(sid == NS - 1)
    def _():
        pltpu.sync_copy(acc_sh.at[pl.ds(roff, RPS_LAST)],
                        acc_out.at[cid].at[pl.ds(roff, RPS_LAST)])
        if with_deg:
            @pl.when(cid == 0)
            def _():
                pltpu.sync_copy(deg_sh.at[pl.ds(roff, RPS_LAST)],
                                deg_out.at[pl.ds(roff, RPS_LAST)])


def _make_agg(with_deg, dc, nbuf):
    out_type = [jax.ShapeDtypeStruct((NC, N, dc), jnp.float32)]
    scratch = [
        pltpu.VMEM((NCHUNK, C), jnp.int32),
        pltpu.VMEM((NCHUNK, C), jnp.int32),
        pltpu.VMEM((nbuf, C, dc), jnp.float32),
        pltpu.VMEM((ZR, dc), jnp.float32),
    ]
    if with_deg:
        out_type.append(jax.ShapeDtypeStruct((N, DW), jnp.float32))
        scratch += [
            pltpu.VMEM((ZR, DW), jnp.float32),
            pltpu.VMEM((C, DW), jnp.float32),
        ]
    scratch.append(pltpu.SemaphoreType.DMA((nbuf,)))
    scratch.append(pltpu.SemaphoreType.DMA((nbuf,)))
    if with_deg:
        scratch.append(pltpu.SemaphoreType.DMA)
    scratch.append(pltpu.VMEM_SHARED((N, dc), jnp.float32))
    if with_deg:
        scratch.append(pltpu.VMEM_SHARED((N, DW), jnp.float32))
    return pl.kernel(
        functools.partial(_agg_body, with_deg, dc, nbuf),
        out_type=tuple(out_type) if len(out_type) > 1 else out_type[0],
        mesh=_mesh,
        scratch_types=scratch,
        compiler_params=pltpu.CompilerParams(use_tc_tiling_on_sc=False),
    )


def _bn_relu(z, g, be):
    mu = jnp.mean(z, axis=0, keepdims=True)
    var = jnp.mean((z - mu) ** 2, axis=0, keepdims=True)
    z = (z - mu) * lax.rsqrt(var + 1e-5) * g + be
    return jnp.maximum(z, 0.0)


def _combine0_body(h, a, dg, ws0, wn0, b0, g0, be0, ws1, b1,
                   h1_out, zs1_out, recip_out):
    deg = dg[:, 0:1]
    recip = jnp.where(deg > 0, 1.0 / jnp.maximum(deg, 1.0), 0.0)
    hn = jnp.concatenate([a[0], a[1]], axis=1) * recip
    z = jnp.dot(h[...], ws0[...], preferred_element_type=jnp.float32)
    z = z + jnp.dot(hn, wn0[...], preferred_element_type=jnp.float32) + b0[...]
    h1 = _bn_relu(jnp.maximum(z, 0.0), g0[...], be0[...])
    h1_out[...] = h1
    zs1_out[...] = jnp.dot(h1, ws1[...],
                           preferred_element_type=jnp.float32) + b1[...]
    recip_out[...] = recip


def _combine1_body(zs1, a, recip, wn1, g1, be1, wn2, ws2, b2,
                   y2_out, zs2_out):
    hn = jnp.concatenate([a[0], a[1]], axis=1) * recip[...]
    z = zs1[...] + jnp.dot(hn, wn1[...], preferred_element_type=jnp.float32)
    h2 = _bn_relu(jnp.maximum(z, 0.0), g1[...], be1[...])
    y2_out[...] = jnp.dot(h2, wn2[...], preferred_element_type=jnp.float32)
    zs2_out[...] = jnp.dot(h2, ws2[...],
                           preferred_element_type=jnp.float32) + b2[...]


def _final_body(zs, a, recip, out):
    z = zs[...] + jnp.concatenate([a[0], a[1]], axis=1) * recip[...]
    hg = jnp.mean(z, axis=0, keepdims=True)
    m = jnp.max(hg, axis=1, keepdims=True)
    e = jnp.exp(hg - m)
    out[...] = e / jnp.sum(e, axis=1, keepdims=True)


_agg_with_deg = _make_agg(True, 64, 2)
_agg_plain = _make_agg(False, 64, 4)
_agg_out = _make_agg(False, 32, 4)

_f32 = jnp.float32
_combine0 = pl.pallas_call(
    _combine0_body,
    out_shape=(jax.ShapeDtypeStruct((N, D_H), _f32),
               jax.ShapeDtypeStruct((N, D_H), _f32),
               jax.ShapeDtypeStruct((N, 1), _f32)))
_combine1 = pl.pallas_call(
    _combine1_body,
    out_shape=(jax.ShapeDtypeStruct((N, D_OUT), _f32),
               jax.ShapeDtypeStruct((N, D_OUT), _f32)))
_final = pl.pallas_call(
    _final_body, out_shape=jax.ShapeDtypeStruct((1, D_OUT), _f32))


def kernel(features, edge_index, Ws0, Wn0, b0, Ws1, Wn1, b1, Ws2, Wn2, b2,
           g0, be0, g1, be1):
    src2 = edge_index[0] * 2
    srcs = jnp.stack([src2, src2 + 1]).reshape(NC, E // C, C)
    dst2d = edge_index[1].reshape(E // C, C)

    acc0, deg = _agg_with_deg(features.reshape(2 * N, 64), srcs, dst2d)
    h1, zs1, recip = _combine0(features, acc0, deg, Ws0, Wn0,
                               b0.reshape(1, D_H), g0.reshape(1, D_H),
                               be0.reshape(1, D_H), Ws1, b1.reshape(1, D_H))

    acc1 = _agg_plain(h1.reshape(2 * N, 64), srcs, dst2d)
    y2, zs2 = _combine1(zs1, acc1, recip, Wn1, g1.reshape(1, D_H),
                        be1.reshape(1, D_H), Wn2, Ws2,
                        b2.reshape(1, D_OUT))

    acc2 = _agg_out(y2.reshape(2 * N, 32), srcs, dst2d)
    return _final(zs2, acc2, recip)


# speedup vs baseline: 13.3967x; 1.0898x over previous
import functools

import jax
import jax.numpy as jnp
from jax import lax
from jax.experimental import pallas as pl
from jax.experimental.pallas import tpu as pltpu
from jax.experimental.pallas import tpu_sc as plsc

N = 10000
E = 320000
D_IN = 128
D_H = 128
D_OUT = 64

NC = 2
NS = 16
EPW = E // NS
C = 125
NCHUNK = EPW // C
RPS = 640
RPS_LAST = N - (NS - 1) * RPS
ZR = 80
DW = 16

_mesh = plsc.VectorSubcoreMesh(core_axis_name="c", subcore_axis_name="s")


def _agg_body(with_deg, dc, nbuf, zr, h_hbm, src_hbm, dst_hbm, *rest):
    if with_deg:
        (acc_out, deg_out, srcv, dstv, rows, zbuf, zbuf_d, onesv,
         gsem, ssem, dsem, acc_sh, deg_sh) = rest
    else:
        acc_out, srcv, dstv, rows, zbuf, gsem, ssem, acc_sh = rest
    cid = lax.axis_index("c")
    sid = lax.axis_index("s")

    nz = jnp.where(sid < NS - 1, RPS // zr, RPS_LAST // zr)
    roff = pl.multiple_of(sid * RPS, 8)

    @pl.loop(0, zr)
    def _(i):
        @pl.loop(0, dc, step=16)
        def _(j):
            zbuf[i, pl.ds(j, 16)] = jnp.zeros((16,), jnp.float32)

    @pl.loop(0, nz)
    def _(k):
        pltpu.sync_copy(zbuf, acc_sh.at[pl.ds(roff + k * zr, zr)])

    if with_deg:
        @pl.loop(0, zr)
        def _(i):
            zbuf_d[i, pl.ds(0, DW)] = jnp.zeros((DW,), jnp.float32)

        @pl.loop(0, C)
        def _(i):
            onesv[i, pl.ds(0, DW)] = jnp.ones((DW,), jnp.float32)

        @pl.loop(0, nz)
        def _(k):
            pltpu.sync_copy(zbuf_d, deg_sh.at[pl.ds(roff + k * zr, zr)])

    plsc.subcore_barrier()

    coff = pl.multiple_of(sid * NCHUNK, 8)
    pltpu.sync_copy(src_hbm.at[cid].at[pl.ds(coff, NCHUNK)], srcv)
    pltpu.sync_copy(dst_hbm.at[pl.ds(coff, NCHUNK)], dstv)

    def gcopy(j, b):
        return pltpu.make_async_copy(h_hbm.at[srcv.at[j]], rows.at[b],
                                     gsem.at[b])

    def scopy(j, b):
        return pltpu.make_async_copy(rows.at[b], acc_sh.at[dstv.at[j]],
                                     ssem.at[b])

    for b in range(nbuf):
        gcopy(b, b).start()

    @pl.loop(0, NCHUNK, step=nbuf)
    def _(j):
        for b in range(nbuf):
            gcopy(j + b, b).wait()
            pltpu.async_copy(rows.at[b], acc_sh.at[dstv.at[j + b]],
                             ssem.at[b], add=True)
            if with_deg:
                @pl.when(lax.rem(j + b, 2) == cid)
                def _(b=b):
                    pltpu.async_copy(onesv, deg_sh.at[dstv.at[j + b]],
                                     dsem, add=True)
        for b in range(nbuf):
            scopy(j + b, b).wait()

            @pl.when(j + nbuf + b < NCHUNK)
            def _(b=b):
                gcopy(j + nbuf + b, b).start()

    if with_deg:
        @pl.loop(0, NCHUNK, step=2)
        def _(j):
            pltpu.make_async_copy(onesv, deg_sh.at[dstv.at[j + cid]],
                                  dsem).wait()

    plsc.subcore_barrier()

    @pl.when(sid < NS - 1)
    def _():
        pltpu.sync_copy(acc_sh.at[pl.ds(roff, RPS)],
                        acc_out.at[cid].at[pl.ds(roff, RPS)])
        if with_deg:
            pltpu.sync_copy(deg_sh.at[pl.ds(roff, RPS)],
                            deg_out.at[cid].at[pl.ds(roff, RPS)])

    @pl.when(sid == NS - 1)
    def _():
        pltpu.sync_copy(acc_sh.at[pl.ds(roff, RPS_LAST)],
                        acc_out.at[cid].at[pl.ds(roff, RPS_LAST)])
        if with_deg:
            pltpu.sync_copy(deg_sh.at[pl.ds(roff, RPS_LAST)],
                            deg_out.at[cid].at[pl.ds(roff, RPS_LAST)])


def _make_agg(with_deg, dc, nbuf, zr=ZR):
    out_type = [jax.ShapeDtypeStruct((NC, N, dc), jnp.float32)]
    scratch = [
        pltpu.VMEM((NCHUNK, C), jnp.int32),
        pltpu.VMEM((NCHUNK, C), jnp.int32),
        pltpu.VMEM((nbuf, C, dc), jnp.float32),
        pltpu.VMEM((zr, dc), jnp.float32),
    ]
    if with_deg:
        out_type.append(jax.ShapeDtypeStruct((NC, N, DW), jnp.float32))
        scratch += [
            pltpu.VMEM((zr, DW), jnp.float32),
            pltpu.VMEM((C, DW), jnp.float32),
        ]
    scratch.append(pltpu.SemaphoreType.DMA((nbuf,)))
    scratch.append(pltpu.SemaphoreType.DMA((nbuf,)))
    if with_deg:
        scratch.append(pltpu.SemaphoreType.DMA)
    scratch.append(pltpu.VMEM_SHARED((N, dc), jnp.float32))
    if with_deg:
        scratch.append(pltpu.VMEM_SHARED((N, DW), jnp.float32))
    return pl.kernel(
        functools.partial(_agg_body, with_deg, dc, nbuf, zr),
        out_type=tuple(out_type) if len(out_type) > 1 else out_type[0],
        mesh=_mesh,
        scratch_types=scratch,
        compiler_params=pltpu.CompilerParams(use_tc_tiling_on_sc=False),
    )


def _bn_relu(z, g, be):
    mu = jnp.mean(z, axis=0, keepdims=True)
    var = jnp.mean((z - mu) ** 2, axis=0, keepdims=True)
    z = (z - mu) * lax.rsqrt(var + 1e-5) * g + be
    return jnp.maximum(z, 0.0)


def _combine0_body(h, a, dg, ws0, wn0, b0, g0, be0, ws1, b1,
                   h1_out, zs1_out, recip_out):
    deg = dg[0, :, 0:1] + dg[1, :, 0:1]
    recip = jnp.where(deg > 0, 1.0 / jnp.maximum(deg, 1.0), 0.0)
    hn = jnp.concatenate([a[0], a[1]], axis=1) * recip
    z = jnp.dot(h[...], ws0[...], preferred_element_type=jnp.float32)
    z = z + jnp.dot(hn, wn0[...], preferred_element_type=jnp.float32) + b0[...]
    h1 = _bn_relu(jnp.maximum(z, 0.0), g0[...], be0[...])
    h1_out[...] = h1
    zs1_out[...] = jnp.dot(h1, ws1[...],
                           preferred_element_type=jnp.float32) + b1[...]
    recip_out[...] = recip


def _combine1_body(zs1, a, recip, wn1, g1, be1, wn2, ws2, b2,
                   y2_out, zs2_out):
    hn = jnp.concatenate([a[0], a[1]], axis=1) * recip[...]
    z = zs1[...] + jnp.dot(hn, wn1[...], preferred_element_type=jnp.float32)
    h2 = _bn_relu(jnp.maximum(z, 0.0), g1[...], be1[...])
    y2_out[...] = jnp.dot(h2, wn2[...], preferred_element_type=jnp.float32)
    zs2_out[...] = jnp.dot(h2, ws2[...],
                           preferred_element_type=jnp.float32) + b2[...]


def _final_body(zs, a, recip, out):
    z = zs[...] + jnp.concatenate([a[0], a[1]], axis=1) * recip[...]
    hg = jnp.mean(z, axis=0, keepdims=True)
    m = jnp.max(hg, axis=1, keepdims=True)
    e = jnp.exp(hg - m)
    out[...] = e / jnp.sum(e, axis=1, keepdims=True)


_agg_with_deg = _make_agg(True, 64, 4, 40)
_agg_plain = _make_agg(False, 64, 4)
_agg_out = _make_agg(False, 32, 4)

_f32 = jnp.float32
_combine0 = pl.pallas_call(
    _combine0_body,
    out_shape=(jax.ShapeDtypeStruct((N, D_H), _f32),
               jax.ShapeDtypeStruct((N, D_H), _f32),
               jax.ShapeDtypeStruct((N, 1), _f32)))
_combine1 = pl.pallas_call(
    _combine1_body,
    out_shape=(jax.ShapeDtypeStruct((N, D_OUT), _f32),
               jax.ShapeDtypeStruct((N, D_OUT), _f32)))
_final = pl.pallas_call(
    _final_body, out_shape=jax.ShapeDtypeStruct((1, D_OUT), _f32))


def kernel(features, edge_index, Ws0, Wn0, b0, Ws1, Wn1, b1, Ws2, Wn2, b2,
           g0, be0, g1, be1):
    src2 = edge_index[0] * 2
    srcs = jnp.stack([src2, src2 + 1]).reshape(NC, E // C, C)
    dst2d = edge_index[1].reshape(E // C, C)

    acc0, deg = _agg_with_deg(features.reshape(2 * N, 64), srcs, dst2d)
    h1, zs1, recip = _combine0(features, acc0, deg, Ws0, Wn0,
                               b0.reshape(1, D_H), g0.reshape(1, D_H),
                               be0.reshape(1, D_H), Ws1, b1.reshape(1, D_H))

    acc1 = _agg_plain(h1.reshape(2 * N, 64), srcs, dst2d)
    y2, zs2 = _combine1(zs1, acc1, recip, Wn1, g1.reshape(1, D_H),
                        be1.reshape(1, D_H), Wn2, Ws2,
                        b2.reshape(1, D_OUT))

    acc2 = _agg_out(y2.reshape(2 * N, 32), srcs, dst2d)
    return _final(zs2, acc2, recip)


# speedup vs baseline: 13.6483x; 1.0188x over previous
import functools

import jax
import jax.numpy as jnp
from jax import lax
from jax.experimental import pallas as pl
from jax.experimental.pallas import tpu as pltpu
from jax.experimental.pallas import tpu_sc as plsc

N = 10000
E = 320000
D_IN = 128
D_H = 128
D_OUT = 64

NC = 2
NS = 16
EPW = E // NS
C = 125
NCHUNK = EPW // C
RPS = 640
RPS_LAST = N - (NS - 1) * RPS
ZR = 80
DW = 16

_mesh = plsc.VectorSubcoreMesh(core_axis_name="c", subcore_axis_name="s")


def _agg_body(with_deg, dc, nbuf, zr, h_hbm, src_hbm, dst_hbm, *rest):
    if with_deg:
        (acc_out, deg_out, srcv, dstv, rows, zbuf, zbuf_d, onesv,
         gsem, ssem, dsem, isem, acc_sh, deg_sh) = rest
    else:
        acc_out, srcv, dstv, rows, zbuf, gsem, ssem, isem, acc_sh = rest
    cid = lax.axis_index("c")
    sid = lax.axis_index("s")

    coff = pl.multiple_of(sid * NCHUNK, 8)
    icopy_s = pltpu.make_async_copy(src_hbm.at[cid].at[pl.ds(coff, NCHUNK)],
                                    srcv, isem)
    icopy_d = pltpu.make_async_copy(dst_hbm.at[pl.ds(coff, NCHUNK)],
                                    dstv, isem)
    icopy_s.start()
    icopy_d.start()

    nz = jnp.where(sid < NS - 1, RPS // zr, RPS_LAST // zr)
    roff = pl.multiple_of(sid * RPS, 8)

    @pl.loop(0, zr)
    def _(i):
        @pl.loop(0, dc, step=16)
        def _(j):
            zbuf[i, pl.ds(j, 16)] = jnp.zeros((16,), jnp.float32)

    @pl.loop(0, nz)
    def _(k):
        pltpu.sync_copy(zbuf, acc_sh.at[pl.ds(roff + k * zr, zr)])

    if with_deg:
        @pl.loop(0, zr)
        def _(i):
            zbuf_d[i, pl.ds(0, DW)] = jnp.zeros((DW,), jnp.float32)

        @pl.loop(0, C)
        def _(i):
            onesv[i, pl.ds(0, DW)] = jnp.ones((DW,), jnp.float32)

        @pl.loop(0, nz)
        def _(k):
            pltpu.sync_copy(zbuf_d, deg_sh.at[pl.ds(roff + k * zr, zr)])

    plsc.subcore_barrier()

    icopy_s.wait()
    icopy_d.wait()

    def gcopy(j, b):
        return pltpu.make_async_copy(h_hbm.at[srcv.at[j]], rows.at[b],
                                     gsem.at[b])

    def scopy(j, b):
        return pltpu.make_async_copy(rows.at[b], acc_sh.at[dstv.at[j]],
                                     ssem.at[b])

    for b in range(nbuf):
        gcopy(b, b).start()

    @pl.loop(0, NCHUNK, step=nbuf)
    def _(j):
        for b in range(nbuf):
            gcopy(j + b, b).wait()
            pltpu.async_copy(rows.at[b], acc_sh.at[dstv.at[j + b]],
                             ssem.at[b], add=True)
            if with_deg:
                @pl.when(lax.rem(j + b, 2) == cid)
                def _(b=b):
                    pltpu.async_copy(onesv, deg_sh.at[dstv.at[j + b]],
                                     dsem, add=True)
        for b in range(nbuf):
            scopy(j + b, b).wait()

            @pl.when(j + nbuf + b < NCHUNK)
            def _(b=b):
                gcopy(j + nbuf + b, b).start()

    if with_deg:
        @pl.loop(0, NCHUNK, step=2)
        def _(j):
            pltpu.make_async_copy(onesv, deg_sh.at[dstv.at[j + cid]],
                                  dsem).wait()

    plsc.subcore_barrier()

    @pl.when(sid < NS - 1)
    def _():
        pltpu.sync_copy(acc_sh.at[pl.ds(roff, RPS)],
                        acc_out.at[cid].at[pl.ds(roff, RPS)])
        if with_deg:
            pltpu.sync_copy(deg_sh.at[pl.ds(roff, RPS)],
                            deg_out.at[cid].at[pl.ds(roff, RPS)])

    @pl.when(sid == NS - 1)
    def _():
        pltpu.sync_copy(acc_sh.at[pl.ds(roff, RPS_LAST)],
                        acc_out.at[cid].at[pl.ds(roff, RPS_LAST)])
        if with_deg:
            pltpu.sync_copy(deg_sh.at[pl.ds(roff, RPS_LAST)],
                            deg_out.at[cid].at[pl.ds(roff, RPS_LAST)])


def _make_agg(with_deg, dc, nbuf, zr=ZR):
    out_type = [jax.ShapeDtypeStruct((NC, N, dc), jnp.float32)]
    scratch = [
        pltpu.VMEM((NCHUNK, C), jnp.int32),
        pltpu.VMEM((NCHUNK, C), jnp.int32),
        pltpu.VMEM((nbuf, C, dc), jnp.float32),
        pltpu.VMEM((zr, dc), jnp.float32),
    ]
    if with_deg:
        out_type.append(jax.ShapeDtypeStruct((NC, N, DW), jnp.float32))
        scratch += [
            pltpu.VMEM((zr, DW), jnp.float32),
            pltpu.VMEM((C, DW), jnp.float32),
        ]
    scratch.append(pltpu.SemaphoreType.DMA((nbuf,)))
    scratch.append(pltpu.SemaphoreType.DMA((nbuf,)))
    if with_deg:
        scratch.append(pltpu.SemaphoreType.DMA)
    scratch.append(pltpu.SemaphoreType.DMA)
    scratch.append(pltpu.VMEM_SHARED((N, dc), jnp.float32))
    if with_deg:
        scratch.append(pltpu.VMEM_SHARED((N, DW), jnp.float32))
    return pl.kernel(
        functools.partial(_agg_body, with_deg, dc, nbuf, zr),
        out_type=tuple(out_type) if len(out_type) > 1 else out_type[0],
        mesh=_mesh,
        scratch_types=scratch,
        compiler_params=pltpu.CompilerParams(use_tc_tiling_on_sc=False),
    )


def _bn_relu(z, g, be):
    mu = jnp.mean(z, axis=0, keepdims=True)
    var = jnp.mean((z - mu) ** 2, axis=0, keepdims=True)
    z = (z - mu) * lax.rsqrt(var + 1e-5) * g + be
    return jnp.maximum(z, 0.0)


def _combine0_body(h, a, dg, ws0, wn0, b0, g0, be0, ws1, b1,
                   h1_out, zs1_out, recip_out):
    deg = dg[0, :, 0:1] + dg[1, :, 0:1]
    recip = jnp.where(deg > 0, 1.0 / jnp.maximum(deg, 1.0), 0.0)
    hn = jnp.concatenate([a[0], a[1]], axis=1) * recip
    z = jnp.dot(h[...], ws0[...], preferred_element_type=jnp.float32)
    z = z + jnp.dot(hn, wn0[...], preferred_element_type=jnp.float32) + b0[...]
    h1 = _bn_relu(jnp.maximum(z, 0.0), g0[...], be0[...])
    h1_out[...] = h1
    zs1_out[...] = jnp.dot(h1, ws1[...],
                           preferred_element_type=jnp.float32) + b1[...]
    recip_out[...] = recip


def _combine1_body(zs1, a, recip, wn1, g1, be1, wn2, ws2, b2,
                   y2_out, zs2_out):
    hn = jnp.concatenate([a[0], a[1]], axis=1) * recip[...]
    z = zs1[...] + jnp.dot(hn, wn1[...], preferred_element_type=jnp.float32)
    h2 = _bn_relu(jnp.maximum(z, 0.0), g1[...], be1[...])
    y2_out[...] = jnp.dot(h2, wn2[...], preferred_element_type=jnp.float32)
    zs2_out[...] = jnp.dot(h2, ws2[...],
                           preferred_element_type=jnp.float32) + b2[...]


def _final_body(zs, a, recip, out):
    z = zs[...] + jnp.concatenate([a[0], a[1]], axis=1) * recip[...]
    hg = jnp.mean(z, axis=0, keepdims=True)
    m = jnp.max(hg, axis=1, keepdims=True)
    e = jnp.exp(hg - m)
    out[...] = e / jnp.sum(e, axis=1, keepdims=True)


_agg_with_deg = _make_agg(True, 64, 4, 40)
_agg_plain = _make_agg(False, 64, 4)
_agg_out = _make_agg(False, 32, 4)

_f32 = jnp.float32
_combine0 = pl.pallas_call(
    _combine0_body,
    out_shape=(jax.ShapeDtypeStruct((N, D_H), _f32),
               jax.ShapeDtypeStruct((N, D_H), _f32),
               jax.ShapeDtypeStruct((N, 1), _f32)))
_combine1 = pl.pallas_call(
    _combine1_body,
    out_shape=(jax.ShapeDtypeStruct((N, D_OUT), _f32),
               jax.ShapeDtypeStruct((N, D_OUT), _f32)))
_final = pl.pallas_call(
    _final_body, out_shape=jax.ShapeDtypeStruct((1, D_OUT), _f32))


def kernel(features, edge_index, Ws0, Wn0, b0, Ws1, Wn1, b1, Ws2, Wn2, b2,
           g0, be0, g1, be1):
    src2 = edge_index[0] * 2
    srcs = jnp.stack([src2, src2 + 1]).reshape(NC, E // C, C)
    dst2d = edge_index[1].reshape(E // C, C)

    acc0, deg = _agg_with_deg(features.reshape(2 * N, 64), srcs, dst2d)
    h1, zs1, recip = _combine0(features, acc0, deg, Ws0, Wn0,
                               b0.reshape(1, D_H), g0.reshape(1, D_H),
                               be0.reshape(1, D_H), Ws1, b1.reshape(1, D_H))

    acc1 = _agg_plain(h1.reshape(2 * N, 64), srcs, dst2d)
    y2, zs2 = _combine1(zs1, acc1, recip, Wn1, g1.reshape(1, D_H),
                        be1.reshape(1, D_H), Wn2, Ws2,
                        b2.reshape(1, D_OUT))

    acc2 = _agg_out(y2.reshape(2 * N, 32), srcs, dst2d)
    return _final(zs2, acc2, recip)
